# baseline (speedup 1.0000x reference)
import functools

import jax
import jax.numpy as jnp
from jax import lax
from jax.experimental import pallas as pl
from jax.experimental.pallas import tpu as pltpu
from jax.experimental.pallas import tpu_sc as plsc

_TV = 50265
_V = 52267
_B = 4
_S = 250
_R = _B * _S
_TOTAL = _R * _V
_NW = 32
_RPW = 32
_BUF = 52272

_TEXT_BLOCKS = 3141
_BOUND_COL = 50256
_BOX_LO = 3142
_BOX_HI = 3266
_TAIL_COL = 52256


def _row_reduce(buf, lane, neg, res, i):
    def maxloop8(col0, iters, accs):
        def body(k, a):
            base = col0 + k * 128
            a0 = jnp.maximum(a[0], buf[pl.ds(base, 16)])
            a1 = jnp.maximum(a[1], buf[pl.ds(base + 16, 16)])
            a2 = jnp.maximum(a[2], buf[pl.ds(base + 32, 16)])
            a3 = jnp.maximum(a[3], buf[pl.ds(base + 48, 16)])
            a0 = jnp.maximum(a0, buf[pl.ds(base + 64, 16)])
            a1 = jnp.maximum(a1, buf[pl.ds(base + 80, 16)])
            a2 = jnp.maximum(a2, buf[pl.ds(base + 96, 16)])
            a3 = jnp.maximum(a3, buf[pl.ds(base + 112, 16)])
            return (a0, a1, a2, a3)
        return lax.fori_loop(0, iters, body, accs)

    def tailblocks(col0, n, acc):
        for t in range(n):
            acc = jnp.maximum(acc, buf[pl.ds(col0 + t * 16, 16)])
        return acc

    accs = maxloop8(0, 392, (neg, neg, neg, neg))
    acc_t = jnp.maximum(jnp.maximum(accs[0], accs[1]),
                        jnp.maximum(accs[2], accs[3]))
    acc_t = tailblocks(392 * 128, 5, acc_t)
    bvec = buf[pl.ds(_BOUND_COL, 16)]
    acc_t = jnp.maximum(acc_t, jnp.where(lane <= 8, bvec, neg))
    m_text = jnp.max(acc_t)
    v_open = jnp.max(jnp.where(lane == 9, bvec, neg))
    v_close = jnp.max(jnp.where(lane == 10, bvec, neg))
    accs = maxloop8(_BOUND_COL + 16, 15, (neg, neg, neg, neg))
    acc_b = jnp.maximum(jnp.maximum(accs[0], accs[1]),
                        jnp.maximum(accs[2], accs[3]))
    acc_b = tailblocks(_BOUND_COL + 16 + 15 * 128, 4, acc_b)
    acc_b = jnp.maximum(acc_b, jnp.where(lane >= 11, bvec, neg))
    tvec = buf[pl.ds(_TAIL_COL, 16)]
    acc_b = jnp.maximum(acc_b, jnp.where(lane <= 10, tvec, neg))
    m_box = jnp.max(acc_b)

    packed = jnp.where(lane == 0, m_text,
             jnp.where(lane == 1, v_open,
             jnp.where(lane == 2, v_close, m_box)))
    res[pl.ds(i * 16, 16)] = packed


def _sc_body(logits, out, buf0, buf1, res, sem0, sem1):
    info = plsc.get_sparse_core_info()
    nc = info.num_cores
    wid = lax.axis_index("s") * nc + lax.axis_index("c")
    lane = lax.broadcasted_iota(jnp.int32, (16,), 0)
    neg = jnp.full((16,), -jnp.inf, jnp.float32)

    def start_dma(i, buf, sem):
        r = jnp.minimum(wid * _RPW + i, _R - 1)
        b = r // _S
        s = r % _S
        pltpu.async_copy(logits.at[b, s], buf.at[pl.ds(0, _V)], sem)

    def wait_dma(buf, sem):
        pltpu.make_async_copy(logits.at[0, 0], buf.at[pl.ds(0, _V)],
                              sem).wait()

    start_dma(0, buf0, sem0)

    def pair_body(j, carry):
        i0 = j * 2
        start_dma(i0 + 1, buf1, sem1)
        wait_dma(buf0, sem0)
        _row_reduce(buf0, lane, neg, res, i0)

        @pl.when(j < _RPW // 2 - 1)
        def _():
            start_dma(i0 + 2, buf0, sem0)

        wait_dma(buf1, sem1)
        _row_reduce(buf1, lane, neg, res, i0 + 1)
        return carry

    lax.fori_loop(0, _RPW // 2, pair_body, 0)
    pltpu.sync_copy(res, out.at[pl.ds(wid * (_RPW * 16), _RPW * 16)])


@functools.partial(jax.jit, static_argnums=())
def _segment_maxes(logits):
    call = pl.kernel(
        _sc_body,
        out_type=jax.ShapeDtypeStruct((_NW * _RPW * 16,), jnp.float32),
        mesh=plsc.VectorSubcoreMesh(core_axis_name="c", subcore_axis_name="s"),
        compiler_params=pltpu.CompilerParams(
            needs_layout_passes=False, use_tc_tiling_on_sc=False),
        scratch_types=[
            pltpu.VMEM((_BUF,), jnp.float32),
            pltpu.VMEM((_BUF,), jnp.float32),
            pltpu.VMEM((_RPW * 16,), jnp.float32),
            pltpu.SemaphoreType.DMA,
            pltpu.SemaphoreType.DMA,
        ],
    )
    return call(logits)


def _tc_body(mt_ref, mo_ref, mc_ref, mb_ref, bt_ref, ts_ref,
             scores_ref, boxes_ref):
    f32 = jnp.float32
    mt = mt_ref[...]
    mo = mo_ref[...]
    mc = mc_ref[...]
    mb = mb_ref[...]

    is_text = mt >= jnp.maximum(jnp.maximum(mo, mc), mb)
    is_open = jnp.logical_not(is_text) & (mo >= jnp.maximum(mc, mb))
    is_close = jnp.logical_not(is_text | is_open) & (mc >= mb)
    is_box = jnp.logical_not(is_text | is_open | is_close)

    ft = is_text.astype(f32)
    fo = is_open.astype(f32)
    fc = is_close.astype(f32)
    fb = is_box.astype(f32)

    jj = lax.broadcasted_iota(jnp.int32, (_S, _S), 0)
    ii = lax.broadcasted_iota(jnp.int32, (_S, _S), 1)
    U = (jj < ii).astype(f32)

    def excl(x):
        return jnp.dot(x, U, preferred_element_type=f32)

    in_bbox = excl(fo - fc)
    box_cnt = excl(fb)
    str_idx = excl(ft)
    total_box = jnp.sum(fb, axis=1, keepdims=True)
    num_rows = jnp.floor((total_box + 3.0) / 4.0)
    box_idx = jnp.floor(box_cnt / 4.0)
    valid = is_text & (in_bbox != 0.0) & (box_idx < num_rows)
    del str_idx

    bi = jnp.where(valid, box_idx, f32(1e6))
    rvec = lax.broadcasted_iota(jnp.int32, (1, 1, 100), 2).astype(f32)
    k = jnp.sum((bi[:, :, None] == rvec).astype(f32), axis=1)

    e1 = jnp.exp(f32(1.0)) - f32(1.0)
    scores_ref[...] = f32(1.0) - f32(1.0) / (f32(256.0) + k * e1)

    ts = ts_ref[...].astype(f32)
    h = ts[:, 0:1]
    w = ts[:, 1:2]
    bt = bt_ref[...]
    xc = bt[:, 0, :]
    yc = bt[:, 1, :]
    bw = bt[:, 2, :]
    bh = bt[:, 3, :]
    boxes_ref[:, 0, :] = (xc - 0.5 * bw) * w
    boxes_ref[:, 1, :] = (yc - 0.5 * bh) * h
    boxes_ref[:, 2, :] = (xc + 0.5 * bw) * w
    boxes_ref[:, 3, :] = (yc + 0.5 * bh) * h


def _post(mt, mo, mc, mb, bt, ts):
    return pl.pallas_call(
        _tc_body,
        out_shape=[
            jax.ShapeDtypeStruct((_B, 100), jnp.float32),
            jax.ShapeDtypeStruct((_B, 4, 100), jnp.float32),
        ],
    )(mt, mo, mc, mb, bt, ts)


def kernel(pred_logits, pred_boxes, target_sizes):
    maxes = _segment_maxes(pred_logits)
    m = maxes.reshape(_NW * _RPW, 16)[:_R, :4].reshape(_B, _S, 4)
    bt = jnp.transpose(pred_boxes, (0, 2, 1))
    scores, boxes_t = _post(m[..., 0], m[..., 1], m[..., 2], m[..., 3],
                            bt, target_sizes)
    boxes = jnp.transpose(boxes_t, (0, 2, 1))
    labels = jnp.ones((_B, 100), jnp.int32)
    return scores, labels, boxes


# speedup vs baseline: 9.0084x; 9.0084x over previous
import functools

import jax
import jax.numpy as jnp
from jax import lax
from jax.experimental import pallas as pl
from jax.experimental.pallas import tpu as pltpu
from jax.experimental.pallas import tpu_sc as plsc

_TV = 50265
_V = 52267
_B = 4
_S = 250
_NW = 32
_NG = _B * 31
_K = 28
_NTEXT = 392
_NTILE = 408


def _sc_body(logits, out, buf0, buf1, res, sem0, sem1):
    info = plsc.get_sparse_core_info()
    nc = info.num_cores
    wid = lax.axis_index("s") * nc + lax.axis_index("c")
    lane = lax.broadcasted_iota(jnp.int32, (16,), 0)
    neg = jnp.full((16,), -jnp.inf, jnp.float32)

    def fire(b, sr, c0, buf, sem, n):
        for t in range(n):
            pltpu.async_copy(
                logits.at[b, pl.ds(sr, 8), pl.ds((c0 + t) * 128, 128)],
                buf.at[t], sem)

    def drain(buf, sem, n):
        for t in range(n):
            pltpu.make_async_copy(
                logits.at[0, pl.ds(0, 8), pl.ds(0, 128)],
                buf.at[t], sem).wait()

    def tile_max(buf, t, su):
        l0 = jnp.maximum(buf[t, su, pl.ds(0, 16)], buf[t, su, pl.ds(16, 16)])
        l1 = jnp.maximum(buf[t, su, pl.ds(32, 16)], buf[t, su, pl.ds(48, 16)])
        l2 = jnp.maximum(buf[t, su, pl.ds(64, 16)], buf[t, su, pl.ds(80, 16)])
        l3 = jnp.maximum(buf[t, su, pl.ds(96, 16)], buf[t, su, pl.ds(112, 16)])
        return jnp.maximum(jnp.maximum(l0, l1), jnp.maximum(l2, l3))

    def reduce_round(buf, lo, hi, accs):
        new = []
        for su in range(8):
            def body(t, a, su=su):
                return jnp.maximum(a, tile_max(buf, t, su))
            new.append(lax.fori_loop(lo, hi, body, accs[su]))
        return tuple(new)

    def group_body(j, carry):
        g = jnp.minimum(wid * 4 + j, _NG - 1)
        b = g // 31
        sr = (g - b * 31) * 8
        fire(b, sr, 0, buf0, sem0, _K)
        accs = (neg,) * 8

        def pair_body(p, accs):
            c0 = p * (2 * _K)
            fire(b, sr, c0 + _K, buf1, sem1, _K)
            drain(buf0, sem0, _K)
            accs = reduce_round(buf0, 0, _K, accs)

            @pl.when(p < 6)
            def _():
                fire(b, sr, c0 + 2 * _K, buf0, sem0, _K)

            @pl.when(p == 6)
            def _():
                fire(b, sr, _NTEXT, buf0, sem0, _NTILE - _NTEXT)

            drain(buf1, sem1, _K)
            return reduce_round(buf1, 0, _K, accs)

        accs = lax.fori_loop(0, 7, pair_body, accs)
        drain(buf0, sem0, _NTILE - _NTEXT)

        baccs = reduce_round(buf0, 1, _NTILE - _NTEXT, (neg,) * 8)
        for su in range(8):
            t_acc = accs[su]
            for k in range(5):
                t_acc = jnp.maximum(t_acc, buf0[0, su, pl.ds(k * 16, 16)])
            bvec = buf0[0, su, pl.ds(80, 16)]
            t_acc = jnp.maximum(t_acc, jnp.where(lane <= 8, bvec, neg))
            m_text = jnp.max(t_acc)
            v_open = jnp.max(jnp.where(lane == 9, bvec, neg))
            v_close = jnp.max(jnp.where(lane == 10, bvec, neg))
            b_acc = jnp.maximum(baccs[su], jnp.where(lane >= 11, bvec, neg))
            b_acc = jnp.maximum(b_acc, buf0[0, su, pl.ds(96, 16)])
            b_acc = jnp.maximum(b_acc, buf0[0, su, pl.ds(112, 16)])
            m_box = jnp.max(b_acc)
            packed = jnp.where(lane == 0, m_text,
                     jnp.where(lane == 1, v_open,
                     jnp.where(lane == 2, v_close, m_box)))
            res[pl.ds((j * 8 + su) * 16, 16)] = packed
        return carry

    lax.fori_loop(0, 4, group_body, 0)
    pltpu.sync_copy(res, out.at[pl.ds(wid * 512, 512)])


@functools.partial(jax.jit, static_argnums=())
def _segment_maxes(logits):
    call = pl.kernel(
        _sc_body,
        out_type=jax.ShapeDtypeStruct((_NW * 512,), jnp.float32),
        mesh=plsc.VectorSubcoreMesh(core_axis_name="c", subcore_axis_name="s"),
        compiler_params=pltpu.CompilerParams(needs_layout_passes=False),
        scratch_types=[
            pltpu.VMEM((_K, 8, 128), jnp.float32),
            pltpu.VMEM((_K, 8, 128), jnp.float32),
            pltpu.VMEM((512,), jnp.float32),
            pltpu.SemaphoreType.DMA,
            pltpu.SemaphoreType.DMA,
        ],
    )
    return call(logits)


def _tc_body(mt_ref, mo_ref, mc_ref, mb_ref, tail_ref, last2_ref,
             bt_ref, ts_ref, scores_ref, boxes_ref):
    f32 = jnp.float32
    mt = mt_ref[...]
    mo = mo_ref[...]
    mc = mc_ref[...]
    mb = mb_ref[...]
    mb = jnp.maximum(mb, jnp.max(tail_ref[...], axis=-1))
    last2 = last2_ref[...]
    mt2 = jnp.max(last2[:, :, :_TV], axis=-1)
    mo2 = last2[:, :, _TV]
    mc2 = last2[:, :, _TV + 1]
    mb2 = jnp.max(last2[:, :, _TV + 2:], axis=-1)
    row = lax.broadcasted_iota(jnp.int32, (_B, _S), 1)

    def inject(base, two):
        return jnp.where(row == _S - 2, two[:, 0:1],
                         jnp.where(row == _S - 1, two[:, 1:2], base))

    mt = inject(mt, mt2)
    mo = inject(mo, mo2)
    mc = inject(mc, mc2)
    mb = inject(mb, mb2)

    is_text = mt >= jnp.maximum(jnp.maximum(mo, mc), mb)
    is_open = jnp.logical_not(is_text) & (mo >= jnp.maximum(mc, mb))
    is_close = jnp.logical_not(is_text | is_open) & (mc >= mb)
    is_box = jnp.logical_not(is_text | is_open | is_close)

    ft = is_text.astype(f32)
    fo = is_open.astype(f32)
    fc = is_close.astype(f32)
    fb = is_box.astype(f32)

    jj = lax.broadcasted_iota(jnp.int32, (_S, _S), 0)
    ii = lax.broadcasted_iota(jnp.int32, (_S, _S), 1)
    U = (jj < ii).astype(f32)

    def excl(x):
        return jnp.dot(x, U, preferred_element_type=f32)

    in_bbox = excl(fo - fc)
    box_cnt = excl(fb)
    total_box = jnp.sum(fb, axis=1, keepdims=True)
    num_rows = jnp.floor((total_box + 3.0) / 4.0)
    box_idx = jnp.floor(box_cnt / 4.0)
    valid = is_text & (in_bbox != 0.0) & (box_idx < num_rows)

    bi = jnp.where(valid, box_idx, f32(1e6))
    rvec = lax.broadcasted_iota(jnp.int32, (1, 1, 100), 2).astype(f32)
    k = jnp.sum((bi[:, :, None] == rvec).astype(f32), axis=1)

    e1 = jnp.exp(f32(1.0)) - f32(1.0)
    scores_ref[...] = f32(1.0) - f32(1.0) / (f32(256.0) + k * e1)

    ts = ts_ref[...].astype(f32)
    h = ts[:, 0:1]
    w = ts[:, 1:2]
    bt = bt_ref[...]
    xc = bt[:, 0, :]
    yc = bt[:, 1, :]
    bw = bt[:, 2, :]
    bh = bt[:, 3, :]
    boxes_ref[:, 0, :] = (xc - 0.5 * bw) * w
    boxes_ref[:, 1, :] = (yc - 0.5 * bh) * h
    boxes_ref[:, 2, :] = (xc + 0.5 * bw) * w
    boxes_ref[:, 3, :] = (yc + 0.5 * bh) * h


def _post(mt, mo, mc, mb, tail, last2, bt, ts):
    return pl.pallas_call(
        _tc_body,
        out_shape=[
            jax.ShapeDtypeStruct((_B, 100), jnp.float32),
            jax.ShapeDtypeStruct((_B, 4, 100), jnp.float32),
        ],
    )(mt, mo, mc, mb, tail, last2, bt, ts)


def kernel(pred_logits, pred_boxes, target_sizes):
    maxes = _segment_maxes(pred_logits)
    m4 = maxes.reshape(_NW, 4, 8, 16)[:31, :, :, :4]
    m4 = m4.reshape(_B, 31 * 8, 4)
    m4 = jnp.concatenate(
        [m4, jnp.zeros((_B, 2, 4), jnp.float32)], axis=1)
    tail = lax.slice(pred_logits, (0, 0, _NTILE * 128), (_B, _S, _V))
    last2 = lax.slice(pred_logits, (0, _S - 2, 0), (_B, _S, _V))
    bt = jnp.transpose(pred_boxes, (0, 2, 1))
    scores, boxes_t = _post(m4[..., 0], m4[..., 1], m4[..., 2], m4[..., 3],
                            tail, last2, bt, target_sizes)
    boxes = jnp.transpose(boxes_t, (0, 2, 1))
    labels = jnp.ones((_B, 100), jnp.int32)
    return scores, labels, boxes
